# baseline (speedup 1.0000x reference)
import functools

import jax
import jax.numpy as jnp
from jax import lax
from jax.experimental import pallas as pl
from jax.experimental.pallas import tpu as pltpu
from jax.experimental.pallas import tpu_sc as plsc

P = 32
E = 50000
F = 1600000
M = 64
N = 65536
PE1 = P * E + 1

VR = F // 128

NC = 2
NS = 16
NW = NC * NS
QPW = N // NW
CH = 64
NCHUNK = QPW // CH
LANES = 16
NG = CH // LANES
PO_OFF = 128 * 128

_MESH = plsc.VectorSubcoreMesh(core_axis_name="c", subcore_axis_name="s")
_PARAMS = pltpu.CompilerParams(needs_layout_passes=False)


def _sc_offsets_one(preds32, bounds32, off_table):
    @functools.partial(
        pl.kernel,
        out_type=[
            jax.ShapeDtypeStruct((N,), jnp.int32),
            jax.ShapeDtypeStruct((N,), jnp.int32),
        ],
        mesh=_MESH,
        compiler_params=_PARAMS,
        scratch_types=[
            pltpu.VMEM((QPW,), jnp.int32),
            pltpu.VMEM((QPW,), jnp.int32),
            pltpu.VMEM((QPW,), jnp.int32),
            pltpu.VMEM((QPW,), jnp.int32),
            pltpu.VMEM((2 * CH,), jnp.int32),
            pltpu.VMEM((2 * CH,), jnp.int32),
            pltpu.VMEM((2 * CH,), jnp.int32),
            pltpu.VMEM((2 * CH,), jnp.int32),
            pltpu.SemaphoreType.DMA,
            pltpu.SemaphoreType.DMA,
        ],
    )
    def body(p_hbm, b_hbm, off_hbm, st_hbm, en_hbm,
             pv, bv, st_acc, en_acc, idx0, idx1, so0, so1, sem0, sem1):
        wid = lax.axis_index("s") * NC + lax.axis_index("c")
        qw = wid * jnp.int32(QPW)
        pltpu.sync_copy(p_hbm.at[pl.ds(qw, QPW)], pv)
        pltpu.sync_copy(b_hbm.at[pl.ds(qw, QPW)], bv)
        idx = (idx0, idx1)
        so = (so0, so1)
        sem = (sem0, sem1)

        def issue(c, par):
            cb = c * jnp.int32(CH)
            for g in range(NG):
                s = pl.ds(cb + g * LANES, LANES)
                k = pv[s] * jnp.int32(E) + bv[s]
                idx[par][pl.ds(g * LANES, LANES)] = k
                idx[par][pl.ds(CH + g * LANES, LANES)] = k + jnp.int32(1)
            pltpu.async_copy(off_hbm.at[idx[par]], so[par], sem[par])

        def wait(par):
            pltpu.make_async_copy(off_hbm.at[idx[par]], so[par],
                                  sem[par]).wait()

        def consume(c, par):
            cb = c * jnp.int32(CH)
            for g in range(NG):
                sl = pl.ds(g * LANES, LANES)
                sh = pl.ds(CH + g * LANES, LANES)
                st_acc[pl.ds(cb + g * LANES, LANES)] = so[par][sl]
                en_acc[pl.ds(cb + g * LANES, LANES)] = so[par][sh]

        issue(jnp.int32(0), 0)

        def pipe(i, carry):
            c0 = jnp.int32(2) * i
            c1 = c0 + jnp.int32(1)
            issue(c1, 1)
            wait(0)
            consume(c0, 0)

            @pl.when(c1 + jnp.int32(1) < jnp.int32(NCHUNK))
            def _():
                issue(c1 + jnp.int32(1), 0)

            wait(1)
            consume(c1, 1)
            return carry

        lax.fori_loop(jnp.int32(0), jnp.int32(NCHUNK // 2), pipe,
                      jnp.int32(0))
        pltpu.sync_copy(st_acc, st_hbm.at[pl.ds(qw, QPW)])
        pltpu.sync_copy(en_acc, en_hbm.at[pl.ds(qw, QPW)])

    return body(preds32, bounds32, off_table)


def _sc_rows(dirs32, starts, psv, pov):
    @functools.partial(
        pl.kernel,
        out_type=jax.ShapeDtypeStruct((M, N), jnp.int32),
        mesh=_MESH,
        compiler_params=_PARAMS,
        scratch_types=[
            pltpu.VMEM((QPW,), jnp.int32),
            pltpu.VMEM((QPW,), jnp.int32),
            pltpu.VMEM((2 * CH,), jnp.int32),
            pltpu.VMEM((2 * CH,), jnp.int32),
            pltpu.VMEM((CH,), jnp.int32),
            pltpu.VMEM((CH,), jnp.int32),
            pltpu.VMEM((CH,), jnp.int32),
            pltpu.VMEM((CH,), jnp.int32),
            pltpu.VMEM((4 * CH, 2 * M), jnp.int32),
            pltpu.VMEM((4 * CH, 2 * M), jnp.int32),
            pltpu.VMEM((M, 2 * CH), jnp.int32),
            pltpu.SemaphoreType.DMA,
            pltpu.SemaphoreType.DMA,
        ],
    )
    def body(d_hbm, st_hbm, psv_hbm, pov_hbm, cand_hbm,
             dv, stv, rows0, rows1, addr0, addr1, alim0, alim1,
             bufs0, bufs1, out_v, semr0, semr1):
        wid = lax.axis_index("s") * NC + lax.axis_index("c")
        qw = wid * jnp.int32(QPW)
        pltpu.sync_copy(d_hbm.at[pl.ds(qw, QPW)], dv)
        pltpu.sync_copy(st_hbm.at[pl.ds(qw, QPW)], stv)
        iota = lax.iota(jnp.int32, LANES)
        rows = (rows0, rows1)
        addr = (addr0, addr1)
        alim = (alim0, alim1)
        bufs = (bufs0, bufs1)
        semr = (semr0, semr1)

        def compute(c, par):
            cb = c * jnp.int32(CH)
            for g in range(NG):
                sl = pl.ds(g * LANES, LANES)
                dmask = dv[pl.ds(cb + g * LANES, LANES)] != jnp.int32(0)
                st = stv[pl.ds(cb + g * LANES, LANES)]
                stc = jnp.minimum(st, jnp.int32(F - 1))
                r0 = lax.shift_right_logical(stc, jnp.int32(7))
                r0c = jnp.minimum(r0, jnp.int32(VR - 2))
                lq = iota + jnp.int32(g * LANES)
                plsc.store_scatter(rows[par], [jnp.int32(2) * lq], r0c)
                plsc.store_scatter(rows[par],
                                   [jnp.int32(2) * lq + jnp.int32(1)],
                                   r0c + jnp.int32(1))
                base = (jnp.where(dmask, jnp.int32(PO_OFF), jnp.int32(0))
                        + lq * jnp.int32(4 * M)
                        - jnp.int32(128) * r0c)
                addr[par][sl] = base + stc
                alim[par][sl] = base + jnp.int32(F - 1)

        def issue_rows(par):
            pltpu.async_copy(psv_hbm.at[rows[par]],
                             bufs[par].at[pl.ds(0, 2 * CH)], semr[par])
            pltpu.async_copy(pov_hbm.at[rows[par]],
                             bufs[par].at[pl.ds(2 * CH, 2 * CH)], semr[par])

        def wait_rows(par):
            pltpu.make_async_copy(psv_hbm.at[rows[par]],
                                  bufs[par].at[pl.ds(0, 2 * CH)],
                                  semr[par]).wait()
            pltpu.make_async_copy(pov_hbm.at[rows[par]],
                                  bufs[par].at[pl.ds(2 * CH, 2 * CH)],
                                  semr[par]).wait()

        def extract(par, colhalf):
            b = bufs[par]
            for g in range(NG):
                lqo = iota + jnp.int32(g * LANES + colhalf * CH)
                al = alim[par][pl.ds(g * LANES, LANES)]

                def jl(j, acc):
                    a, cj = acc
                    r = lax.shift_right_logical(a, jnp.int32(7))
                    col = lax.bitwise_and(a, jnp.int32(127))
                    vals = plsc.load_gather(b, [r, col])
                    plsc.store_scatter(out_v, [cj, lqo], vals)
                    return (jnp.minimum(a + jnp.int32(1), al),
                            cj + jnp.int32(1))

                lax.fori_loop(0, M, jl,
                              (addr[par][pl.ds(g * LANES, LANES)],
                               jnp.zeros((LANES,), jnp.int32)),
                              unroll=8)

        def out_dma(c_low):
            start = pl.multiple_of(qw + c_low * jnp.int32(CH), 2 * CH)
            pltpu.sync_copy(out_v, cand_hbm.at[:, pl.ds(start, 2 * CH)])

        compute(jnp.int32(0), 0)
        issue_rows(0)

        def pipe(i, carry):
            c0 = jnp.int32(2) * i
            c1 = c0 + jnp.int32(1)
            compute(c1, 1)
            issue_rows(1)

            @pl.when(i > jnp.int32(0))
            def _():
                out_dma(c0 - jnp.int32(2))

            wait_rows(0)
            extract(0, 0)

            @pl.when(c1 + jnp.int32(1) < jnp.int32(NCHUNK))
            def _():
                compute(c1 + jnp.int32(1), 0)
                issue_rows(0)

            wait_rows(1)
            extract(1, 1)
            return carry

        lax.fori_loop(jnp.int32(0), jnp.int32(NCHUNK // 2), pipe,
                      jnp.int32(0))
        out_dma(jnp.int32(NCHUNK - 2))

    return body(dirs32, starts, psv, pov)


def kernel(preds, bound_args, direction, ps_sorted_objs, ps_offsets,
           po_sorted_subjs, po_offsets):
    p32 = preds.astype(jnp.int32)
    b32 = bound_args.astype(jnp.int32)
    d32 = direction.astype(jnp.int32)
    pso32 = ps_offsets.astype(jnp.int32)
    st_ps, en_ps = _sc_offsets_one(p32, b32, pso32)
    poo32 = po_offsets.astype(jnp.int32)
    st_po, en_po = _sc_offsets_one(p32, b32, poo32)
    psv = ps_sorted_objs.astype(jnp.int32).reshape(VR, 128)
    pov = po_sorted_subjs.astype(jnp.int32).reshape(VR, 128)
    use_po = d32 != 0
    starts = jnp.where(use_po, st_po, st_ps)
    ends = jnp.where(use_po, en_po, en_ps)
    counts = jnp.minimum(jnp.maximum(ends - starts, 0), M).astype(jnp.int32)
    candT = _sc_rows(d32, starts, psv, pov)
    candidates = candT.T.astype(jnp.uint32).astype(jnp.int64)
    valid = jnp.arange(M, dtype=jnp.int32)[None, :] < counts[:, None]
    return candidates, valid


# speedup vs baseline: 1.0068x; 1.0068x over previous
import functools

import jax
import jax.numpy as jnp
from jax import lax
from jax.experimental import pallas as pl
from jax.experimental.pallas import tpu as pltpu
from jax.experimental.pallas import tpu_sc as plsc

P = 32
E = 50000
F = 1600000
M = 64
N = 65536
PE1 = P * E + 1

VR = F // 128

NC = 2
NS = 16
NW = NC * NS
QPW = N // NW
CH = 64
NCHUNK = QPW // CH
LANES = 16
NG = CH // LANES
PO_OFF = 128 * 128

_MESH = plsc.VectorSubcoreMesh(core_axis_name="c", subcore_axis_name="s")
_PARAMS = pltpu.CompilerParams(needs_layout_passes=False)


def _sc_offsets(preds32, bounds32, dirs32, ps_off32, po_off32):
    @functools.partial(
        pl.kernel,
        out_type=[
            jax.ShapeDtypeStruct((N,), jnp.int32),
            jax.ShapeDtypeStruct((N,), jnp.int32),
        ],
        mesh=_MESH,
        compiler_params=_PARAMS,
        scratch_types=[
            pltpu.VMEM((QPW,), jnp.int32),
            pltpu.VMEM((QPW,), jnp.int32),
            pltpu.VMEM((QPW,), jnp.int32),
            pltpu.VMEM((QPW,), jnp.int32),
            pltpu.VMEM((QPW,), jnp.int32),
            pltpu.VMEM((2 * CH,), jnp.int32),
            pltpu.VMEM((2 * CH,), jnp.int32),
            pltpu.VMEM((2 * CH,), jnp.int32),
            pltpu.VMEM((2 * CH,), jnp.int32),
            pltpu.VMEM((2 * CH,), jnp.int32),
            pltpu.VMEM((2 * CH,), jnp.int32),
            pltpu.SemaphoreType.DMA,
            pltpu.SemaphoreType.DMA,
        ],
    )
    def body(p_hbm, b_hbm, d_hbm, pso_hbm, poo_hbm, st_hbm, cnt_hbm,
             pv, bv, dv, st_acc, cnt_acc,
             idx0, idx1, sops0, sops1, sopo0, sopo1, sem0, sem1):
        wid = lax.axis_index("s") * NC + lax.axis_index("c")
        qw = wid * jnp.int32(QPW)
        pltpu.sync_copy(p_hbm.at[pl.ds(qw, QPW)], pv)
        pltpu.sync_copy(b_hbm.at[pl.ds(qw, QPW)], bv)
        pltpu.sync_copy(d_hbm.at[pl.ds(qw, QPW)], dv)
        idx = (idx0, idx1)
        sops = (sops0, sops1)
        sopo = (sopo0, sopo1)
        sem = (sem0, sem1)

        def issue(c, par):
            cb = c * jnp.int32(CH)
            for g in range(NG):
                s = pl.ds(cb + g * LANES, LANES)
                k = pv[s] * jnp.int32(E) + bv[s]
                idx[par][pl.ds(g * LANES, LANES)] = k
                idx[par][pl.ds(CH + g * LANES, LANES)] = k + jnp.int32(1)
            pltpu.async_copy(pso_hbm.at[idx[par]], sops[par], sem[par])
            pltpu.async_copy(poo_hbm.at[idx[par]], sopo[par], sem[par])

        def wait(par):
            pltpu.make_async_copy(pso_hbm.at[idx[par]], sops[par],
                                  sem[par]).wait()
            pltpu.make_async_copy(poo_hbm.at[idx[par]], sopo[par],
                                  sem[par]).wait()

        def consume(c, par):
            cb = c * jnp.int32(CH)
            for g in range(NG):
                sl = pl.ds(g * LANES, LANES)
                sh = pl.ds(CH + g * LANES, LANES)
                dmask = dv[pl.ds(cb + g * LANES, LANES)] != jnp.int32(0)
                st = jnp.where(dmask, sopo[par][sl], sops[par][sl])
                en = jnp.where(dmask, sopo[par][sh], sops[par][sh])
                cnt = jnp.minimum(jnp.maximum(en - st, jnp.int32(0)),
                                  jnp.int32(M))
                st_acc[pl.ds(cb + g * LANES, LANES)] = st
                cnt_acc[pl.ds(cb + g * LANES, LANES)] = cnt

        issue(jnp.int32(0), 0)

        def pipe(i, carry):
            c0 = jnp.int32(2) * i
            c1 = c0 + jnp.int32(1)
            issue(c1, 1)
            wait(0)
            consume(c0, 0)

            @pl.when(c1 + jnp.int32(1) < jnp.int32(NCHUNK))
            def _():
                issue(c1 + jnp.int32(1), 0)

            wait(1)
            consume(c1, 1)
            return carry

        lax.fori_loop(jnp.int32(0), jnp.int32(NCHUNK // 2), pipe,
                      jnp.int32(0))
        pltpu.sync_copy(st_acc, st_hbm.at[pl.ds(qw, QPW)])
        pltpu.sync_copy(cnt_acc, cnt_hbm.at[pl.ds(qw, QPW)])

    return body(preds32, bounds32, dirs32, ps_off32, po_off32)


def _sc_rows(dirs32, starts, psv, pov):
    @functools.partial(
        pl.kernel,
        out_type=jax.ShapeDtypeStruct((M, N), jnp.int32),
        mesh=_MESH,
        compiler_params=_PARAMS,
        scratch_types=[
            pltpu.VMEM((QPW,), jnp.int32),
            pltpu.VMEM((QPW,), jnp.int32),
            pltpu.VMEM((2 * CH,), jnp.int32),
            pltpu.VMEM((2 * CH,), jnp.int32),
            pltpu.VMEM((CH,), jnp.int32),
            pltpu.VMEM((CH,), jnp.int32),
            pltpu.VMEM((CH,), jnp.int32),
            pltpu.VMEM((CH,), jnp.int32),
            pltpu.VMEM((4 * CH, 2 * M), jnp.int32),
            pltpu.VMEM((4 * CH, 2 * M), jnp.int32),
            pltpu.VMEM((M, 2 * CH), jnp.int32),
            pltpu.SemaphoreType.DMA,
            pltpu.SemaphoreType.DMA,
        ],
    )
    def body(d_hbm, st_hbm, psv_hbm, pov_hbm, cand_hbm,
             dv, stv, rows0, rows1, addr0, addr1, alim0, alim1,
             bufs0, bufs1, out_v, semr0, semr1):
        wid = lax.axis_index("s") * NC + lax.axis_index("c")
        qw = wid * jnp.int32(QPW)
        pltpu.sync_copy(d_hbm.at[pl.ds(qw, QPW)], dv)
        pltpu.sync_copy(st_hbm.at[pl.ds(qw, QPW)], stv)
        iota = lax.iota(jnp.int32, LANES)
        rows = (rows0, rows1)
        addr = (addr0, addr1)
        alim = (alim0, alim1)
        bufs = (bufs0, bufs1)
        semr = (semr0, semr1)

        def compute(c, par):
            cb = c * jnp.int32(CH)
            for g in range(NG):
                sl = pl.ds(g * LANES, LANES)
                dmask = dv[pl.ds(cb + g * LANES, LANES)] != jnp.int32(0)
                st = stv[pl.ds(cb + g * LANES, LANES)]
                stc = jnp.minimum(st, jnp.int32(F - 1))
                r0 = lax.shift_right_logical(stc, jnp.int32(7))
                r0c = jnp.minimum(r0, jnp.int32(VR - 2))
                lq = iota + jnp.int32(g * LANES)
                plsc.store_scatter(rows[par], [jnp.int32(2) * lq], r0c)
                plsc.store_scatter(rows[par],
                                   [jnp.int32(2) * lq + jnp.int32(1)],
                                   r0c + jnp.int32(1))
                base = (jnp.where(dmask, jnp.int32(PO_OFF), jnp.int32(0))
                        + lq * jnp.int32(4 * M)
                        - jnp.int32(128) * r0c)
                addr[par][sl] = base + stc
                alim[par][sl] = base + jnp.int32(F - 1)

        def issue_rows(par):
            pltpu.async_copy(psv_hbm.at[rows[par]],
                             bufs[par].at[pl.ds(0, 2 * CH)], semr[par])
            pltpu.async_copy(pov_hbm.at[rows[par]],
                             bufs[par].at[pl.ds(2 * CH, 2 * CH)], semr[par])

        def wait_rows(par):
            pltpu.make_async_copy(psv_hbm.at[rows[par]],
                                  bufs[par].at[pl.ds(0, 2 * CH)],
                                  semr[par]).wait()
            pltpu.make_async_copy(pov_hbm.at[rows[par]],
                                  bufs[par].at[pl.ds(2 * CH, 2 * CH)],
                                  semr[par]).wait()

        def extract(par, colhalf):
            b = bufs[par]
            for g in range(NG):
                lqo = iota + jnp.int32(g * LANES + colhalf * CH)
                al = alim[par][pl.ds(g * LANES, LANES)]

                def jl(j, acc):
                    a, cj = acc
                    r = lax.shift_right_logical(a, jnp.int32(7))
                    col = lax.bitwise_and(a, jnp.int32(127))
                    vals = plsc.load_gather(b, [r, col])
                    plsc.store_scatter(out_v, [cj, lqo], vals)
                    return (jnp.minimum(a + jnp.int32(1), al),
                            cj + jnp.int32(1))

                lax.fori_loop(0, M, jl,
                              (addr[par][pl.ds(g * LANES, LANES)],
                               jnp.zeros((LANES,), jnp.int32)),
                              unroll=8)

        def out_dma(c_low):
            start = pl.multiple_of(qw + c_low * jnp.int32(CH), 2 * CH)
            pltpu.sync_copy(out_v, cand_hbm.at[:, pl.ds(start, 2 * CH)])

        compute(jnp.int32(0), 0)
        issue_rows(0)

        def pipe(i, carry):
            c0 = jnp.int32(2) * i
            c1 = c0 + jnp.int32(1)
            compute(c1, 1)
            issue_rows(1)

            @pl.when(i > jnp.int32(0))
            def _():
                out_dma(c0 - jnp.int32(2))

            wait_rows(0)
            extract(0, 0)

            @pl.when(c1 + jnp.int32(1) < jnp.int32(NCHUNK))
            def _():
                compute(c1 + jnp.int32(1), 0)
                issue_rows(0)

            wait_rows(1)
            extract(1, 1)
            return carry

        lax.fori_loop(jnp.int32(0), jnp.int32(NCHUNK // 2), pipe,
                      jnp.int32(0))
        out_dma(jnp.int32(NCHUNK - 2))

    return body(dirs32, starts, psv, pov)


def kernel(preds, bound_args, direction, ps_sorted_objs, ps_offsets,
           po_sorted_subjs, po_offsets):
    p32 = preds.astype(jnp.int32)
    b32 = bound_args.astype(jnp.int32)
    d32 = direction.astype(jnp.int32)
    pso32 = ps_offsets.astype(jnp.int32)
    poo32 = po_offsets.astype(jnp.int32)
    starts, counts = _sc_offsets(p32, b32, d32, pso32, poo32)
    psv = ps_sorted_objs.astype(jnp.int32).reshape(VR, 128)
    pov = po_sorted_subjs.astype(jnp.int32).reshape(VR, 128)
    candT = _sc_rows(d32, starts, psv, pov)
    candidates = candT.T.astype(jnp.uint32).astype(jnp.int64)
    valid = jnp.arange(M, dtype=jnp.int32)[None, :] < counts[:, None]
    return candidates, valid
